# baseline (speedup 1.0000x reference)
import functools

import jax
import jax.numpy as jnp
from jax import lax
from jax.experimental import pallas as pl
from jax.experimental.pallas import tpu as pltpu
from jax.experimental.pallas import tpu_sc as plsc

N = 262144
C = 21
K = 1024
HIST = 2 * C * K
B = 2048
NW = 32
PIECES = 1
NP = N // PIECES
CHUNK = 11264
NCH = 11 * NP // (NW * CHUNK)


def _binize_body(x_ref, lab_ref, out_ref):
    l = x_ref[...].T
    m = jnp.max(l, axis=0, keepdims=True)
    ex = jnp.exp(l - m)
    s = jnp.sum(ex, axis=0, keepdims=True)
    p = ex / s
    lab = lab_ref[0]
    cls = lax.broadcasted_iota(jnp.int32, (C, B), 0)
    fgm = lab == cls
    e = jnp.where(fgm, p, 1.0 - p)
    b = jnp.clip((e * K).astype(jnp.int32), 0, K - 1)
    idx_cnt = b + cls * K
    e_fg = jnp.sum(jnp.where(fgm, e, 0.0), axis=0, keepdims=True)
    b_fg = jnp.clip((e_fg * K).astype(jnp.int32), 0, K - 1)
    idx_fg = b_fg + lab * K + C * K
    x22 = jnp.concatenate([idx_cnt, idx_fg], axis=0)
    out_ref[...] = x22[:11] | (x22[11:] << 16)


def _hist_body(idx_hbm, out_hbm, buf0, buf1, hist_v, sem0, sem1):
    wid = lax.axis_index("s") * 2 + lax.axis_index("c")

    def zero_step(i, _):
        hist_v[pl.ds(i * 16, 16)] = jnp.zeros((16,), jnp.float32)
        return 0

    lax.fori_loop(0, HIST // 16, zero_step, 0, unroll=8)

    ones = jnp.ones((16,), jnp.float32)
    mask16 = jnp.full((16,), 0xFFFF, jnp.int32)

    bufs = (buf0, buf1)
    sems = (sem0, sem1)
    base = wid * NCH
    pltpu.make_async_copy(idx_hbm.at[base], buf0, sem0).start()
    for j in range(NCH):
        buf = bufs[j % 2]
        sem = sems[j % 2]
        if j + 1 < NCH:
            pltpu.make_async_copy(
                idx_hbm.at[base + j + 1], bufs[(j + 1) % 2], sems[(j + 1) % 2]
            ).start()
        pltpu.make_async_copy(idx_hbm.at[base + j], buf, sem).wait()

        def scat_step(i, _):
            v = buf[pl.ds(i * 16, 16)]
            lo = v & mask16
            hi = lax.shift_right_logical(v, 16)
            plsc.addupdate_scatter(hist_v, [lo], ones)
            plsc.addupdate_scatter(hist_v, [hi], ones)
            return 0

        lax.fori_loop(0, CHUNK // 16, scat_step, 0, unroll=8)
    pltpu.sync_copy(hist_v, out_hbm.at[wid])


def _hist_call(idx):
    call = functools.partial(
        pl.kernel,
        mesh=plsc.VectorSubcoreMesh(core_axis_name="c", subcore_axis_name="s"),
        compiler_params=pltpu.CompilerParams(needs_layout_passes=False),
        out_type=jax.ShapeDtypeStruct((NW, HIST), jnp.float32),
        scratch_types=[
            pltpu.VMEM((CHUNK,), jnp.int32),
            pltpu.VMEM((CHUNK,), jnp.int32),
            pltpu.VMEM((HIST,), jnp.float32),
            pltpu.SemaphoreType.DMA,
            pltpu.SemaphoreType.DMA,
        ],
    )(_hist_body)
    return call(idx)


def _finish_body(*refs):
    out_ref = refs[-1]
    s = jnp.sum(refs[0][...], axis=0)
    for r in refs[1:-1]:
        s = s + jnp.sum(r[...], axis=0)
    cnt = s[:C]
    fgc = s[C:]
    G = jnp.sum(fgc, axis=1, keepdims=True)
    row = lax.broadcasted_iota(jnp.int32, (K, K), 0)
    col = lax.broadcasted_iota(jnp.int32, (K, K), 1)
    tri = (row >= col).astype(jnp.float32)
    n = jnp.dot(cnt, tri, preferred_element_type=jnp.float32)
    f = jnp.dot(fgc, tri, preferred_element_type=jnp.float32)
    jac = n / jnp.maximum(G + n - f, 1.0)
    jac_next = jnp.concatenate([jac[:, 1:], jnp.zeros((C, 1), jnp.float32)], axis=1)
    v = (lax.broadcasted_iota(jnp.int32, (C, K), 1).astype(jnp.float32) + 0.5) * (1.0 / K)
    loss = jnp.sum(v * (jac - jac_next), axis=1, keepdims=True)
    present = (G > 0).astype(jnp.float32)
    total = jnp.sum(loss * present) / jnp.maximum(jnp.sum(present), 1.0)
    out_ref[...] = jnp.reshape(total, (1, 1))


def kernel(logits, labels):
    lab32 = labels.astype(jnp.int32)
    hist_parts = []
    for pc in range(PIECES):
        xs = logits[pc * NP:(pc + 1) * NP]
        lab3 = lab32[pc * NP:(pc + 1) * NP].reshape(NP // B, 1, B)
        idx = pl.pallas_call(
            _binize_body,
            grid=(NP // B,),
            in_specs=[
                pl.BlockSpec((B, C), lambda i: (i, 0)),
                pl.BlockSpec((1, 1, B), lambda i: (i, 0, 0)),
            ],
            out_specs=pl.BlockSpec((11, B), lambda i: (0, i)),
            out_shape=jax.ShapeDtypeStruct((11, NP), jnp.int32),
        )(xs, lab3)
        hists = _hist_call(idx.reshape(NW * NCH, CHUNK))
        hist_parts.append(hists.reshape(NW, 2 * C, K))

    out = pl.pallas_call(
        _finish_body,
        out_shape=jax.ShapeDtypeStruct((1, 1), jnp.float32),
    )(*hist_parts)
    return out[0, 0]


# speedup vs baseline: 1.4915x; 1.4915x over previous
import functools

import jax
import jax.numpy as jnp
from jax import lax
from jax.experimental import pallas as pl
from jax.experimental.pallas import tpu as pltpu
from jax.experimental.pallas import tpu_sc as plsc

N = 262144
C = 21
K = 1024
HIST = 2 * C * K
B = 2048
NW = 32
PIECES = 1
NP = N // PIECES
CHUNK = 11264
NCH = 11 * NP // (NW * CHUNK)


def _binize_body(lt_ref, lab_ref, out_ref):
    l = lt_ref[...].astype(jnp.float32)
    m = jnp.max(l, axis=0, keepdims=True)
    ex = jnp.exp(l - m)
    s = jnp.sum(ex, axis=0, keepdims=True)
    p = ex / s
    lab = lab_ref[0]
    cls = lax.broadcasted_iota(jnp.int32, (C, B), 0)
    fgm = lab == cls
    e = jnp.where(fgm, p, 1.0 - p)
    b = jnp.clip((e * K).astype(jnp.int32), 0, K - 1)
    idx_cnt = b + cls * K
    e_fg = jnp.sum(jnp.where(fgm, e, 0.0), axis=0, keepdims=True)
    b_fg = jnp.clip((e_fg * K).astype(jnp.int32), 0, K - 1)
    idx_fg = b_fg + lab * K + C * K
    x22 = jnp.concatenate([idx_cnt, idx_fg], axis=0)
    out_ref[...] = x22[:11] | (x22[11:] << 16)


def _hist_body(idx_hbm, out_hbm, buf0, buf1, hist_v, sem0, sem1):
    wid = lax.axis_index("s") * 2 + lax.axis_index("c")

    def zero_step(i, _):
        hist_v[pl.ds(i * 16, 16)] = jnp.zeros((16,), jnp.float32)
        return 0

    lax.fori_loop(0, HIST // 16, zero_step, 0, unroll=8)

    ones = jnp.ones((16,), jnp.float32)
    mask16 = jnp.full((16,), 0xFFFF, jnp.int32)

    bufs = (buf0, buf1)
    sems = (sem0, sem1)
    base = wid * NCH
    pltpu.make_async_copy(idx_hbm.at[base], buf0, sem0).start()
    for j in range(NCH):
        buf = bufs[j % 2]
        sem = sems[j % 2]
        if j + 1 < NCH:
            pltpu.make_async_copy(
                idx_hbm.at[base + j + 1], bufs[(j + 1) % 2], sems[(j + 1) % 2]
            ).start()
        pltpu.make_async_copy(idx_hbm.at[base + j], buf, sem).wait()

        def scat_step(i, _):
            v = buf[pl.ds(i * 16, 16)]
            lo = v & mask16
            hi = lax.shift_right_logical(v, 16)
            plsc.addupdate_scatter(hist_v, [lo], ones)
            plsc.addupdate_scatter(hist_v, [hi], ones)
            return 0

        lax.fori_loop(0, CHUNK // 16, scat_step, 0, unroll=8)
    pltpu.sync_copy(hist_v, out_hbm.at[wid])


def _hist_call(idx):
    call = functools.partial(
        pl.kernel,
        mesh=plsc.VectorSubcoreMesh(core_axis_name="c", subcore_axis_name="s"),
        compiler_params=pltpu.CompilerParams(needs_layout_passes=False),
        out_type=jax.ShapeDtypeStruct((NW, HIST), jnp.float32),
        scratch_types=[
            pltpu.VMEM((CHUNK,), jnp.int32),
            pltpu.VMEM((CHUNK,), jnp.int32),
            pltpu.VMEM((HIST,), jnp.float32),
            pltpu.SemaphoreType.DMA,
            pltpu.SemaphoreType.DMA,
        ],
    )(_hist_body)
    return call(idx)


def _finish_body(*refs):
    out_ref = refs[-1]
    s = jnp.sum(refs[0][...], axis=0)
    for r in refs[1:-1]:
        s = s + jnp.sum(r[...], axis=0)
    cnt = s[:C]
    fgc = s[C:]
    G = jnp.sum(fgc, axis=1, keepdims=True)
    row = lax.broadcasted_iota(jnp.int32, (K, K), 0)
    col = lax.broadcasted_iota(jnp.int32, (K, K), 1)
    tri = (row >= col).astype(jnp.float32)
    n = jnp.dot(cnt, tri, preferred_element_type=jnp.float32)
    f = jnp.dot(fgc, tri, preferred_element_type=jnp.float32)
    jac = n / jnp.maximum(G + n - f, 1.0)
    jac_next = jnp.concatenate([jac[:, 1:], jnp.zeros((C, 1), jnp.float32)], axis=1)
    v = (lax.broadcasted_iota(jnp.int32, (C, K), 1).astype(jnp.float32) + 0.5) * (1.0 / K)
    loss = jnp.sum(v * (jac - jac_next), axis=1, keepdims=True)
    present = (G > 0).astype(jnp.float32)
    total = jnp.sum(loss * present) / jnp.maximum(jnp.sum(present), 1.0)
    out_ref[...] = jnp.reshape(total, (1, 1))


def kernel(logits, labels):
    lab32 = labels.astype(jnp.int32)
    hist_parts = []
    for pc in range(PIECES):
        lt = logits[pc * NP:(pc + 1) * NP].astype(jnp.bfloat16).T
        lab3 = lab32[pc * NP:(pc + 1) * NP].reshape(NP // B, 1, B)
        idx = pl.pallas_call(
            _binize_body,
            grid=(NP // B,),
            in_specs=[
                pl.BlockSpec((C, B), lambda i: (0, i)),
                pl.BlockSpec((1, 1, B), lambda i: (i, 0, 0)),
            ],
            out_specs=pl.BlockSpec((11, B), lambda i: (0, i)),
            out_shape=jax.ShapeDtypeStruct((11, NP), jnp.int32),
        )(lt, lab3)
        hists = _hist_call(idx.reshape(NW * NCH, CHUNK))
        hist_parts.append(hists.reshape(NW, 2 * C, K))

    out = pl.pallas_call(
        _finish_body,
        out_shape=jax.ShapeDtypeStruct((1, 1), jnp.float32),
    )(*hist_parts)
    return out[0, 0]


# speedup vs baseline: 1.5868x; 1.0639x over previous
import functools

import jax
import jax.numpy as jnp
from jax import lax
from jax.experimental import pallas as pl
from jax.experimental.pallas import tpu as pltpu
from jax.experimental.pallas import tpu_sc as plsc

N = 262144
C = 21
K = 1024
HIST = 2 * C * K
B = 2048
NW = 32
PIECES = 1
NP = N // PIECES
CHUNK = 11264
NCH = 11 * NP // (NW * CHUNK)


def _binize_body(lt_ref, lab_ref, out_ref):
    l = lt_ref[...]
    m = jnp.max(l, axis=0, keepdims=True)
    ex = jnp.exp(l - m)
    s = jnp.sum(ex, axis=0, keepdims=True)
    p = ex / s
    lab = lab_ref[0]
    cls = lax.broadcasted_iota(jnp.int32, (C, B), 0)
    fgm = lab == cls
    e = jnp.where(fgm, p, 1.0 - p)
    b = jnp.clip((e * K).astype(jnp.int32), 0, K - 1)
    idx_cnt = b + cls * K
    e_fg = jnp.sum(jnp.where(fgm, e, 0.0), axis=0, keepdims=True)
    b_fg = jnp.clip((e_fg * K).astype(jnp.int32), 0, K - 1)
    idx_fg = b_fg + lab * K + C * K
    x22 = jnp.concatenate([idx_cnt, idx_fg], axis=0)
    out_ref[...] = x22[:11] | (x22[11:] << 16)


def _hist_body(idx_hbm, out_hbm, buf0, buf1, hist_v, sem0, sem1):
    wid = lax.axis_index("s") * 2 + lax.axis_index("c")

    def zero_step(i, _):
        hist_v[pl.ds(i * 16, 16)] = jnp.zeros((16,), jnp.float32)
        return 0

    lax.fori_loop(0, HIST // 16, zero_step, 0, unroll=8)

    ones = jnp.ones((16,), jnp.float32)
    mask16 = jnp.full((16,), 0xFFFF, jnp.int32)

    bufs = (buf0, buf1)
    sems = (sem0, sem1)
    base = wid * NCH
    pltpu.make_async_copy(idx_hbm.at[base], buf0, sem0).start()
    for j in range(NCH):
        buf = bufs[j % 2]
        sem = sems[j % 2]
        if j + 1 < NCH:
            pltpu.make_async_copy(
                idx_hbm.at[base + j + 1], bufs[(j + 1) % 2], sems[(j + 1) % 2]
            ).start()
        pltpu.make_async_copy(idx_hbm.at[base + j], buf, sem).wait()

        def scat_step(i, _):
            v = buf[pl.ds(i * 16, 16)]
            lo = v & mask16
            hi = lax.shift_right_logical(v, 16)
            plsc.addupdate_scatter(hist_v, [lo], ones)
            plsc.addupdate_scatter(hist_v, [hi], ones)
            return 0

        lax.fori_loop(0, CHUNK // 16, scat_step, 0, unroll=8)
    pltpu.sync_copy(hist_v, out_hbm.at[wid])


def _hist_call(idx):
    call = functools.partial(
        pl.kernel,
        mesh=plsc.VectorSubcoreMesh(core_axis_name="c", subcore_axis_name="s"),
        compiler_params=pltpu.CompilerParams(needs_layout_passes=False),
        out_type=jax.ShapeDtypeStruct((NW, HIST), jnp.float32),
        scratch_types=[
            pltpu.VMEM((CHUNK,), jnp.int32),
            pltpu.VMEM((CHUNK,), jnp.int32),
            pltpu.VMEM((HIST,), jnp.float32),
            pltpu.SemaphoreType.DMA,
            pltpu.SemaphoreType.DMA,
        ],
    )(_hist_body)
    return call(idx)


def _finish_body(*refs):
    out_ref = refs[-1]
    s = jnp.sum(refs[0][...], axis=0)
    for r in refs[1:-1]:
        s = s + jnp.sum(r[...], axis=0)
    cnt = s[:C]
    fgc = s[C:]
    G = jnp.sum(fgc, axis=1, keepdims=True)
    row = lax.broadcasted_iota(jnp.int32, (K, K), 0)
    col = lax.broadcasted_iota(jnp.int32, (K, K), 1)
    tri = (row >= col).astype(jnp.float32)
    n = jnp.dot(cnt, tri, preferred_element_type=jnp.float32)
    f = jnp.dot(fgc, tri, preferred_element_type=jnp.float32)
    jac = n / jnp.maximum(G + n - f, 1.0)
    jac_next = jnp.concatenate([jac[:, 1:], jnp.zeros((C, 1), jnp.float32)], axis=1)
    v = (lax.broadcasted_iota(jnp.int32, (C, K), 1).astype(jnp.float32) + 0.5) * (1.0 / K)
    loss = jnp.sum(v * (jac - jac_next), axis=1, keepdims=True)
    present = (G > 0).astype(jnp.float32)
    total = jnp.sum(loss * present) / jnp.maximum(jnp.sum(present), 1.0)
    out_ref[...] = jnp.reshape(total, (1, 1))


def kernel(logits, labels):
    lab32 = labels.astype(jnp.int32)
    hist_parts = []
    for pc in range(PIECES):
        lt = logits[pc * NP:(pc + 1) * NP].T
        lab3 = lab32[pc * NP:(pc + 1) * NP].reshape(NP // B, 1, B)
        idx = pl.pallas_call(
            _binize_body,
            grid=(NP // B,),
            in_specs=[
                pl.BlockSpec((C, B), lambda i: (0, i)),
                pl.BlockSpec((1, 1, B), lambda i: (i, 0, 0)),
            ],
            out_specs=pl.BlockSpec((11, B), lambda i: (0, i)),
            out_shape=jax.ShapeDtypeStruct((11, NP), jnp.int32),
        )(lt, lab3)
        hists = _hist_call(idx.reshape(NW * NCH, CHUNK))
        hist_parts.append(hists.reshape(NW, 2 * C, K))

    out = pl.pallas_call(
        _finish_body,
        out_shape=jax.ShapeDtypeStruct((1, 1), jnp.float32),
    )(*hist_parts)
    return out[0, 0]
